# baseline (speedup 1.0000x reference)
import functools

import jax
import jax.numpy as jnp
from jax import lax
from jax.experimental import pallas as pl
from jax.experimental.pallas import tpu as pltpu
from jax.experimental.pallas import tpu_sc as plsc

N_BITS = 20
OUT_DIM = 16
BATCH = 16384
NROWS = 1 << N_BITS

NUM_CORES = 2
NUM_SUBCORES = 16
LANES = 16
NUM_WORKERS = NUM_CORES * NUM_SUBCORES
BPW = BATCH // NUM_WORKERS
BLK = 128
NBLK = BPW // BLK
GRP = BLK // LANES
BLKW = BLK * OUT_DIM
HALF_WORDS = (NROWS // 128) * 1024


def _code_body(imgt_ref, v_ref, s_ref):
    a = imgt_ref[...]
    pw = jnp.left_shift(
        jnp.int32(1), lax.broadcasted_iota(jnp.int32, (N_BITS, 1), 0)
    )
    v_ref[...] = jnp.sum(jnp.where(a > 0, pw, jnp.int32(0)), axis=0)
    s_ref[...] = jnp.sum(jnp.abs(a), axis=0) * (1.0 / N_BITS)


_TCB = BATCH // 2
_code = pl.pallas_call(
    _code_body,
    grid=(2,),
    in_specs=[pl.BlockSpec((N_BITS, _TCB), lambda i: (0, i))],
    out_specs=[
        pl.BlockSpec((_TCB,), lambda i: (i,)),
        pl.BlockSpec((_TCB,), lambda i: (i,)),
    ],
    out_shape=[
        jax.ShapeDtypeStruct((BATCH,), jnp.int32),
        jax.ShapeDtypeStruct((BATCH,), jnp.float32),
    ],
)


def _gather_body(
    v_hbm, scale_hbm, enc_hbm, out_hbm,
    v_v, scale_v, idx_v, outt_v, sem_arr, wsem,
):
    sems = [sem_arr.at[i] for i in range(NBLK)]
    wid = lax.axis_index("s") * NUM_CORES + lax.axis_index("c")
    base = wid * BPW

    pltpu.sync_copy(v_hbm.at[pl.ds(base, BPW)], v_v)
    pltpu.sync_copy(scale_hbm.at[pl.ds(base, BPW)], scale_v)

    copies = []
    for blk in range(NBLK):
        def build_fn(g, carry, blk=blk):
            vv = v_v[pl.ds(blk * BLK + g * LANES, LANES)]
            common = ((vv >> 7) << 10) + (vv & 127)
            for d in range(OUT_DIM):
                i, r = d // 8, d % 8
                idx_v[blk, pl.ds((i * 8 + r) * BLK + g * LANES, LANES)] = (
                    common + (i * HALF_WORDS + r * 128)
                )
            return carry
        lax.fori_loop(0, GRP, build_fn, 0)
        copies.append(
            pltpu.async_copy(
                enc_hbm.at[idx_v.at[blk]], outt_v.at[blk], sems[blk]
            )
        )

    wcopies = []
    for blk in range(NBLK):
        copies[blk].wait()

        svs = [
            scale_v[pl.ds(blk * BLK + g * LANES, LANES)] for g in range(GRP)
        ]

        def mul_fn(d, carry, blk=blk, svs=svs):
            for g in range(GRP):
                sl = pl.ds(d * BLK + g * LANES, LANES)
                outt_v[blk, sl] = outt_v[blk, sl] * svs[g]
            return carry
        lax.fori_loop(0, OUT_DIM, mul_fn, 0)

        for i in range(2):
            src = outt_v.at[blk, pl.ds(i * 8 * BLK, 8 * BLK)]
            dst0 = (i * (BATCH // 128) + wid * NBLK + blk) * 1024
            wcopies.append(
                pltpu.async_copy(src, out_hbm.at[pl.ds(dst0, 8 * BLK)], wsem)
            )
    for cp in wcopies:
        cp.wait()


_gather = functools.partial(
    pl.kernel,
    out_type=jax.ShapeDtypeStruct((2 * (BATCH // 128) * 1024,), jnp.float32),
    mesh=plsc.VectorSubcoreMesh(core_axis_name="c", subcore_axis_name="s"),
    compiler_params=pltpu.CompilerParams(
        needs_layout_passes=False, use_tc_tiling_on_sc=False
    ),
    scratch_types=[
        pltpu.VMEM((BPW,), jnp.int32),
        pltpu.VMEM((BPW,), jnp.float32),
        pltpu.VMEM((NBLK, BLKW), jnp.int32),
        pltpu.VMEM((NBLK, BLKW), jnp.float32),
        pltpu.SemaphoreType.DMA((NBLK,)),
        pltpu.SemaphoreType.DMA,
    ],
)(_gather_body)


def kernel(image, encoding):
    enc_flat = (
        encoding.reshape(NROWS // 128, 128, 2, 8)
        .transpose(2, 0, 3, 1)
        .reshape(-1)
    )
    v, scale = _code(image.T)
    outt = _gather(v, scale, enc_flat)
    return (
        outt.reshape(2, BATCH // 128, 8, 128)
        .transpose(1, 3, 0, 2)
        .reshape(BATCH, OUT_DIM)
    )


# speedup vs baseline: 1.0015x; 1.0015x over previous
import functools

import jax
import jax.numpy as jnp
from jax import lax
from jax.experimental import pallas as pl
from jax.experimental.pallas import tpu as pltpu
from jax.experimental.pallas import tpu_sc as plsc

N_BITS = 20
OUT_DIM = 16
BATCH = 16384
NROWS = 1 << N_BITS

NUM_CORES = 2
NUM_SUBCORES = 16
LANES = 16
NUM_WORKERS = NUM_CORES * NUM_SUBCORES
BPW = BATCH // NUM_WORKERS
BLK = 128
NBLK = BPW // BLK
GRP = BLK // LANES
BLKW = BLK * OUT_DIM
HALF_WORDS = (NROWS // 128) * 1024


def _code_body(imgt_ref, v_ref, s_ref):
    a = imgt_ref[...]
    pw = jnp.left_shift(
        jnp.int32(1), lax.broadcasted_iota(jnp.int32, (N_BITS, 1), 0)
    )
    v_ref[...] = jnp.sum(jnp.where(a > 0, pw, jnp.int32(0)), axis=0)
    s_ref[...] = jnp.sum(jnp.abs(a), axis=0) * (1.0 / N_BITS)


_code = pl.pallas_call(
    _code_body,
    out_shape=[
        jax.ShapeDtypeStruct((BATCH,), jnp.int32),
        jax.ShapeDtypeStruct((BATCH,), jnp.float32),
    ],
)


def _gather_body(
    v_hbm, scale_hbm, enc_hbm, out_hbm,
    v_v, scale_v, idx_v, outt_v, sem_arr, wsem,
):
    sems = [sem_arr.at[i] for i in range(NBLK)]
    wid = lax.axis_index("s") * NUM_CORES + lax.axis_index("c")
    base = wid * BPW

    pltpu.sync_copy(v_hbm.at[pl.ds(base, BPW)], v_v)
    pltpu.sync_copy(scale_hbm.at[pl.ds(base, BPW)], scale_v)

    copies = []
    for blk in range(NBLK):
        def build_fn(g, carry, blk=blk):
            vv = v_v[pl.ds(blk * BLK + g * LANES, LANES)]
            common = ((vv >> 7) << 10) + (vv & 127)
            for d in range(OUT_DIM):
                i, r = d // 8, d % 8
                idx_v[blk, pl.ds((i * 8 + r) * BLK + g * LANES, LANES)] = (
                    common + (i * HALF_WORDS + r * 128)
                )
            return carry
        lax.fori_loop(0, GRP, build_fn, 0)
        copies.append(
            pltpu.async_copy(
                enc_hbm.at[idx_v.at[blk]], outt_v.at[blk], sems[blk]
            )
        )

    wcopies = []
    for blk in range(NBLK):
        copies[blk].wait()

        svs = [
            scale_v[pl.ds(blk * BLK + g * LANES, LANES)] for g in range(GRP)
        ]

        def mul_fn(d, carry, blk=blk, svs=svs):
            for g in range(GRP):
                sl = pl.ds(d * BLK + g * LANES, LANES)
                outt_v[blk, sl] = outt_v[blk, sl] * svs[g]
            return carry
        lax.fori_loop(0, OUT_DIM, mul_fn, 0)

        for i in range(2):
            src = outt_v.at[blk, pl.ds(i * 8 * BLK, 8 * BLK)]
            dst0 = (i * (BATCH // 128) + wid * NBLK + blk) * 1024
            wcopies.append(
                pltpu.async_copy(src, out_hbm.at[pl.ds(dst0, 8 * BLK)], wsem)
            )
    for cp in wcopies:
        cp.wait()


_gather = functools.partial(
    pl.kernel,
    out_type=jax.ShapeDtypeStruct((2 * (BATCH // 128) * 1024,), jnp.float32),
    mesh=plsc.VectorSubcoreMesh(core_axis_name="c", subcore_axis_name="s"),
    compiler_params=pltpu.CompilerParams(
        needs_layout_passes=False, use_tc_tiling_on_sc=False
    ),
    scratch_types=[
        pltpu.VMEM((BPW,), jnp.int32),
        pltpu.VMEM((BPW,), jnp.float32),
        pltpu.VMEM((NBLK, BLKW), jnp.int32),
        pltpu.VMEM((NBLK, BLKW), jnp.float32),
        pltpu.SemaphoreType.DMA((NBLK,)),
        pltpu.SemaphoreType.DMA,
    ],
)(_gather_body)


def kernel(image, encoding):
    enc_flat = (
        encoding.reshape(NROWS // 128, 128, 2, 8)
        .transpose(2, 0, 3, 1)
        .reshape(-1)
    )
    v, scale = _code(image.T)
    outt = _gather(v, scale, enc_flat)
    return (
        outt.reshape(2, BATCH // 128, 8, 128)
        .transpose(1, 3, 0, 2)
        .reshape(BATCH, OUT_DIM)
    )
